# Initial kernel scaffold; baseline (speedup 1.0000x reference)
import jax
import jax.numpy as jnp
from jax.experimental import pallas as pl


def kernel(node_features, edge_index, edge_features, W_node_w, W_node_b, W_query_w, W_query_b, W_key_w, W_key_b, W_value_w, W_value_b, W_edge_w, W_edge_b, attn1_w, attn1_b, attn2_w, attn2_b, ln_gamma, ln_beta):
    raise NotImplementedError("write your pallas kernel here")


# speedup vs baseline: 2.4773x; 2.4773x over previous
import functools

import jax
import jax.numpy as jnp
from jax import lax
from jax.experimental import pallas as pl
from jax.experimental.pallas import tpu as pltpu
from jax.experimental.pallas import tpu_sc as plsc

N, E, D, DE, H = 10000, 320000, 128, 16, 8
NC, NS = 2, 16
NW = NC * NS
EPW = E // NW
CHUNK = 200
NCHUNK = EPW // CHUNK
EBLK = 2560
NEBLK = E // EBLK
RBLK = 400
NRBLK = N // RBLK
NPAD = 10240
NPT = NPAD // NS
ZROWS = 128

_HI = jax.lax.Precision.HIGHEST


def _dot(a, b, dims):
    return lax.dot_general(a, b, (dims, ((), ())), precision=_HI,
                           preferred_element_type=jnp.float32)


def _lrelu(x):
    return jnp.where(x >= 0, x, 0.2 * x)


def _node_tables_body(x_ref, wq_ref, wk_ref, wv_ref, bv_ref,
                      xs_ref, xt_ref, v_ref):
    x = x_ref[...]
    xs_ref[...] = _dot(x, wq_ref[...], ((1,), (0,)))
    xt_ref[...] = _dot(x, wk_ref[...], ((1,), (0,)))
    v_ref[...] = _dot(x, wv_ref[...], ((1,), (0,))) + bv_ref[...]


def _node_tables(x, wq, wk, wv, bv):
    blk = lambda i: (i, 0)
    full = lambda i: (0, 0)
    return pl.pallas_call(
        _node_tables_body,
        grid=(NRBLK,),
        in_specs=[
            pl.BlockSpec((RBLK, D), blk),
            pl.BlockSpec((D, D), full),
            pl.BlockSpec((D, D), full),
            pl.BlockSpec((D, D), full),
            pl.BlockSpec((1, D), full),
        ],
        out_specs=[
            pl.BlockSpec((RBLK, D), blk),
            pl.BlockSpec((RBLK, D), blk),
            pl.BlockSpec((RBLK, D), blk),
        ],
        out_shape=[jax.ShapeDtypeStruct((N, D), jnp.float32)] * 3,
    )(x, wq, wk, wv, bv)


@functools.cache
def _sc_mesh():
    return plsc.VectorSubcoreMesh(core_axis_name="c", subcore_axis_name="s",
                                  num_cores=NC, num_subcores=NS)


def _edge_gather(src, tgt, xs, xt, v):
    k = pl.kernel(
        _edge_gather_body,
        mesh=_sc_mesh(),
        out_type=(jax.ShapeDtypeStruct((E, D), jnp.float32),
                  jax.ShapeDtypeStruct((E, D), jnp.float32)),
        scratch_types=[
            pltpu.VMEM((CHUNK,), jnp.int32),
            pltpu.VMEM((CHUNK,), jnp.int32),
            pltpu.VMEM((CHUNK, D), jnp.float32),
            pltpu.VMEM((CHUNK, D), jnp.float32),
            pltpu.VMEM((CHUNK, D), jnp.float32),
            pltpu.SemaphoreType.DMA,
        ],
    )
    return k(src, tgt, xs, xt, v)


def _edge_gather_body(src_hbm, tgt_hbm, xs_hbm, xt_hbm, vtab_hbm, g_hbm, vt_hbm,
                      idx_s, idx_t, rows_a, rows_b, rows_v, sem):
    wid = lax.axis_index("s") * NC + lax.axis_index("c")
    base0 = wid * EPW

    def step(ci, carry):
        base = base0 + ci * CHUNK
        pltpu.sync_copy(src_hbm.at[pl.ds(base, CHUNK)], idx_s)
        pltpu.sync_copy(tgt_hbm.at[pl.ds(base, CHUNK)], idx_t)
        pltpu.async_copy(xs_hbm.at[idx_s], rows_a, sem).wait()
        pltpu.async_copy(xt_hbm.at[idx_t], rows_b, sem).wait()
        pltpu.async_copy(vtab_hbm.at[idx_t], rows_v, sem).wait()

        def add_row(e, c2):
            for g in range(D // 16):
                sl = pl.ds(g * 16, 16)
                rows_a[e, sl] = rows_a[e, sl] + rows_b[e, sl]
            return c2

        lax.fori_loop(0, CHUNK, add_row, 0)
        pltpu.sync_copy(rows_a, g_hbm.at[pl.ds(base, CHUNK)])
        pltpu.sync_copy(rows_v, vt_hbm.at[pl.ds(base, CHUNK)])
        return carry

    lax.fori_loop(0, NCHUNK, step, 0)


def _scores_body(g_ref, ef_ref, we_ref, bias_ref, a2w_ref, a2b_ref,
                 score_ref, c_ref, mz_ref):
    i = pl.program_id(0)
    t = g_ref[...] + _dot(ef_ref[...], we_ref[...], ((1,), (0,))) + bias_ref[...]
    a = _lrelu(t)
    s2 = _lrelu(_dot(a, a2w_ref[...], ((1,), (1,))) + a2b_ref[...])
    score = jnp.mean(s2, axis=1, keepdims=True)
    score_ref[...] = score

    m_old = jnp.where(i == 0, -1e30, mz_ref[0])
    z_old = jnp.where(i == 0, 0.0, mz_ref[1])
    m_blk = jnp.max(score)
    m_new = jnp.maximum(m_old, m_blk)
    z_new = z_old * jnp.exp(m_old - m_new) + jnp.sum(jnp.exp(score - m_new))
    mz_ref[0] = m_new
    mz_ref[1] = z_new
    c_ref[0, 0] = m_new + jnp.log(z_new)


def _scores(g, ef, we, bias, a2w, a2b):
    blk = lambda i: (i, 0)
    full = lambda i: (0, 0)
    return pl.pallas_call(
        _scores_body,
        grid=(NEBLK,),
        in_specs=[
            pl.BlockSpec((EBLK, D), blk),
            pl.BlockSpec((EBLK, DE), blk),
            pl.BlockSpec((DE, D), full),
            pl.BlockSpec((1, D), full),
            pl.BlockSpec((H, D), full),
            pl.BlockSpec((1, H), full),
        ],
        out_specs=[
            pl.BlockSpec((EBLK, 1), blk),
            pl.BlockSpec(memory_space=pltpu.SMEM, block_shape=(1, 1),
                         index_map=full),
        ],
        out_shape=[jax.ShapeDtypeStruct((E, 1), jnp.float32),
                   jax.ShapeDtypeStruct((1, 1), jnp.float32)],
        scratch_shapes=[pltpu.SMEM((2,), jnp.float32)],
        compiler_params=pltpu.CompilerParams(
            dimension_semantics=("arbitrary",)),
    )(g, ef, we, bias, a2w, a2b)


def _weigh_body(score_ref, c_ref, vt_ref, y_ref):
    p = jnp.exp(score_ref[...] - c_ref[0, 0])
    y_ref[...] = p * vt_ref[...]


def _weigh(score, c, vt):
    blk = lambda i: (i, 0)
    return pl.pallas_call(
        _weigh_body,
        grid=(NEBLK,),
        in_specs=[
            pl.BlockSpec((EBLK, 1), blk),
            pl.BlockSpec(memory_space=pltpu.SMEM, block_shape=(1, 1),
                         index_map=lambda i: (0, 0)),
            pl.BlockSpec((EBLK, D), blk),
        ],
        out_specs=pl.BlockSpec((EBLK, D), blk),
        out_shape=jax.ShapeDtypeStruct((E, D), jnp.float32),
    )(score, c, vt)


def _scatter(src, y):
    k = pl.kernel(
        _scatter_body,
        mesh=_sc_mesh(),
        out_type=jax.ShapeDtypeStruct((NC, NPAD, D), jnp.float32),
        scratch_types=[
            pltpu.VMEM((CHUNK,), jnp.int32),
            pltpu.VMEM((CHUNK, D), jnp.float32),
            pltpu.VMEM((ZROWS, D), jnp.float32),
            pltpu.VMEM_SHARED((NPAD, D), jnp.float32),
            pltpu.SemaphoreType.DMA,
        ],
    )
    return k(src, y)


def _scatter_body(src_hbm, y_hbm, out_hbm, idx_s, rows, zbuf, acc, sem):
    cid = lax.axis_index("c")
    sid = lax.axis_index("s")
    wid = sid * NC + cid

    def zrow(r, carry):
        for g in range(D // 16):
            zbuf[r, pl.ds(g * 16, 16)] = jnp.zeros((16,), jnp.float32)
        return carry

    lax.fori_loop(0, ZROWS, zrow, 0)
    for j in range(NPT // ZROWS):
        pltpu.sync_copy(zbuf, acc.at[pl.ds(sid * NPT + j * ZROWS, ZROWS)])
    plsc.subcore_barrier()

    def step(ci, carry):
        base = wid * EPW + ci * CHUNK
        pltpu.sync_copy(src_hbm.at[pl.ds(base, CHUNK)], idx_s)
        pltpu.sync_copy(y_hbm.at[pl.ds(base, CHUNK)], rows)
        pltpu.sync_copy(rows, acc.at[idx_s], add=True)
        return carry

    lax.fori_loop(0, NCHUNK, step, 0)
    plsc.subcore_barrier()
    pltpu.sync_copy(acc.at[pl.ds(sid * NPT, NPT)],
                    out_hbm.at[cid, pl.ds(sid * NPT, NPT)])


def _final_body(x_ref, part_ref, wn_ref, bn_ref, gam_ref, bet_ref, out_ref):
    x = _dot(x_ref[...], wn_ref[...], ((1,), (0,))) + bn_ref[...]
    x = x + part_ref[0] + part_ref[1]
    mu = jnp.mean(x, axis=1, keepdims=True)
    xc = x - mu
    var = jnp.mean(xc * xc, axis=1, keepdims=True)
    xn = xc * lax.rsqrt(var + 1e-5)
    out_ref[...] = xn * gam_ref[...] + bet_ref[...]


def _final(x, part, wn, bn, gam, bet):
    blk = lambda i: (i, 0)
    full = lambda i: (0, 0)
    return pl.pallas_call(
        _final_body,
        grid=(NRBLK,),
        in_specs=[
            pl.BlockSpec((RBLK, D), blk),
            pl.BlockSpec((NC, RBLK, D), lambda i: (0, i, 0)),
            pl.BlockSpec((D, D), full),
            pl.BlockSpec((1, D), full),
            pl.BlockSpec((1, D), full),
            pl.BlockSpec((1, D), full),
        ],
        out_specs=pl.BlockSpec((RBLK, D), blk),
        out_shape=jax.ShapeDtypeStruct((N, D), jnp.float32),
    )(x, part, wn, bn, gam, bet)


def kernel(node_features, edge_index, edge_features, W_node_w, W_node_b,
           W_query_w, W_query_b, W_key_w, W_key_b, W_value_w, W_value_b,
           W_edge_w, W_edge_b, attn1_w, attn1_b, attn2_w, attn2_b,
           ln_gamma, ln_beta):
    src = edge_index[0]
    tgt = edge_index[1]

    a1t = attn1_w.T
    aq, ak, ae = a1t[:D], a1t[D:2 * D], a1t[2 * D:]
    wq = W_query_w.T @ aq
    wk = W_key_w.T @ ak
    we = W_edge_w.T @ ae
    bias_tot = (attn1_b + W_query_b @ aq + W_key_b @ ak + W_edge_b @ ae)

    xs, xt, v = _node_tables(node_features, wq, wk, W_value_w.T,
                             W_value_b.reshape(1, D))
    g, vt = _edge_gather(src, tgt, xs, xt, v)
    score, c = _scores(g, edge_features, we, bias_tot.reshape(1, D),
                       attn2_w, attn2_b.reshape(1, H))
    y = _weigh(score, c, vt)
    part = _scatter(src, y)
    return _final(node_features, part, W_node_w.T, W_node_b.reshape(1, D),
                  ln_gamma.reshape(1, D), ln_beta.reshape(1, D))


# speedup vs baseline: 3.1295x; 1.2633x over previous
import functools

import jax
import jax.numpy as jnp
from jax import lax
from jax.experimental import pallas as pl
from jax.experimental.pallas import tpu as pltpu
from jax.experimental.pallas import tpu_sc as plsc

N, E, D, DE, H = 10000, 320000, 128, 16, 8
NC, NS = 2, 16
NW = NC * NS
EPW = E // NW
BCH = 200
NBCH = EPW // BCH
DCH = 80
NDCH = EPW // DCH
EBLK = 2560
NEBLK = E // EBLK
RBLK = 400
NRBLK = N // RBLK
NPAD = 10240
NPT = NPAD // NS
ZROWS = 16

_HI = jax.lax.Precision.HIGHEST


def _dot(a, b, dims):
    return lax.dot_general(a, b, (dims, ((), ())), precision=_HI,
                           preferred_element_type=jnp.float32)


def _lrelu(x):
    return jnp.maximum(x, 0.2 * x)


def _node_tables_body(x_ref, wq_ref, wk_ref, wv_ref, bv_ref,
                      xs_ref, xt_ref, v_ref):
    x = x_ref[...]
    xs_ref[...] = _dot(x, wq_ref[...], ((1,), (0,)))
    xt_ref[...] = _dot(x, wk_ref[...], ((1,), (0,)))
    v_ref[...] = _dot(x, wv_ref[...], ((1,), (0,))) + bv_ref[...]


def _node_tables(x, wq, wk, wv, bv):
    blk = lambda i: (i, 0)
    full = lambda i: (0, 0)
    return pl.pallas_call(
        _node_tables_body,
        grid=(NRBLK,),
        in_specs=[
            pl.BlockSpec((RBLK, D), blk),
            pl.BlockSpec((D, D), full),
            pl.BlockSpec((D, D), full),
            pl.BlockSpec((D, D), full),
            pl.BlockSpec((1, D), full),
        ],
        out_specs=[
            pl.BlockSpec((RBLK, D), blk),
            pl.BlockSpec((RBLK, D), blk),
            pl.BlockSpec((RBLK, D), blk),
        ],
        out_shape=[jax.ShapeDtypeStruct((N, D), jnp.float32)] * 3,
    )(x, wq, wk, wv, bv)


@functools.cache
def _sc_mesh():
    return plsc.VectorSubcoreMesh(core_axis_name="c", subcore_axis_name="s",
                                  num_cores=NC, num_subcores=NS)


def _edge_gather(src, tgt, xs, xt):
    k = pl.kernel(
        _edge_gather_body,
        mesh=_sc_mesh(),
        out_type=jax.ShapeDtypeStruct((E, D), jnp.float32),
        scratch_types=[
            pltpu.VMEM((EPW,), jnp.int32),
            pltpu.VMEM((EPW,), jnp.int32),
            pltpu.VMEM((BCH, D), jnp.float32),
            pltpu.VMEM((BCH, D), jnp.float32),
            pltpu.VMEM((BCH, D), jnp.float32),
            pltpu.VMEM((BCH, D), jnp.float32),
            pltpu.SemaphoreType.DMA,
            pltpu.SemaphoreType.DMA,
            pltpu.SemaphoreType.DMA,
            pltpu.SemaphoreType.DMA,
        ],
    )
    return k(src, tgt, xs, xt)


def _edge_gather_body(src_hbm, tgt_hbm, xs_hbm, xt_hbm, g_hbm,
                      idx_s, idx_t, ra0, rb0, ra1, rb1,
                      s0, s1, s2, s3):
    wid = lax.axis_index("s") * NC + lax.axis_index("c")
    base0 = wid * EPW
    abuf = (ra0, ra1)
    bbuf = (rb0, rb1)
    asem = (s0, s1)
    bsem = (s2, s3)

    pltpu.sync_copy(src_hbm.at[pl.ds(base0, EPW)], idx_s)
    pltpu.sync_copy(tgt_hbm.at[pl.ds(base0, EPW)], idx_t)

    def issue(ci, b):
        sl = pl.ds(ci * BCH, BCH)
        pltpu.async_copy(xs_hbm.at[idx_s.at[sl]], abuf[b], asem[b])
        pltpu.async_copy(xt_hbm.at[idx_t.at[sl]], bbuf[b], bsem[b])

    def wait(b):
        pltpu.make_async_copy(xs_hbm.at[idx_s.at[pl.ds(0, BCH)]],
                              abuf[b], asem[b]).wait()
        pltpu.make_async_copy(xt_hbm.at[idx_t.at[pl.ds(0, BCH)]],
                              bbuf[b], bsem[b]).wait()

    issue(0, 0)

    def step(i, carry):
        for b in range(2):
            ci = 2 * i + b

            @pl.when(ci + 1 < NBCH)
            def _():
                issue(ci + 1, (b + 1) % 2)

            wait(b)

            def add_row(e, c2):
                for g in range(D // 16):
                    sl = pl.ds(g * 16, 16)
                    abuf[b][e, sl] = abuf[b][e, sl] + bbuf[b][e, sl]
                return c2

            lax.fori_loop(0, BCH, add_row, 0)
            pltpu.sync_copy(abuf[b], g_hbm.at[pl.ds(base0 + ci * BCH, BCH)])
        return carry

    lax.fori_loop(0, NBCH // 2, step, 0)


def _scores_body(g_ref, ef_ref, we_ref, bias_ref, a2w_ref, b2_ref,
                 score_ref, c_ref, mz_ref):
    i = pl.program_id(0)
    t = g_ref[...] + _dot(ef_ref[...], we_ref[...], ((1,), (0,))) + bias_ref[...]
    a = _lrelu(t)
    st = _dot(a2w_ref[...], a, ((1,), (1,)))
    st = _lrelu(st + jnp.broadcast_to(b2_ref[:, 0:1], (H, EBLK)))
    srow = jnp.mean(st, axis=0, keepdims=True)
    score_ref[...] = srow[None]

    m_old = jnp.where(i == 0, -1e30, mz_ref[0])
    z_old = jnp.where(i == 0, 0.0, mz_ref[1])
    m_blk = jnp.max(srow)
    m_new = jnp.maximum(m_old, m_blk)
    z_new = z_old * jnp.exp(m_old - m_new) + jnp.sum(jnp.exp(srow - m_new))
    mz_ref[0] = m_new
    mz_ref[1] = z_new
    c_ref[0, 0] = m_new + jnp.log(z_new)


def _scores(g, ef, we, bias, a2w, b2f):
    blk = lambda i: (i, 0)
    full = lambda i: (0, 0)
    return pl.pallas_call(
        _scores_body,
        grid=(NEBLK,),
        in_specs=[
            pl.BlockSpec((EBLK, D), blk),
            pl.BlockSpec((EBLK, DE), blk),
            pl.BlockSpec((DE, D), full),
            pl.BlockSpec((1, D), full),
            pl.BlockSpec((H, D), full),
            pl.BlockSpec((H, D), full),
        ],
        out_specs=[
            pl.BlockSpec((1, 1, EBLK), lambda i: (i, 0, 0)),
            pl.BlockSpec(memory_space=pltpu.SMEM, block_shape=(1, 1),
                         index_map=full),
        ],
        out_shape=[jax.ShapeDtypeStruct((NEBLK, 1, EBLK), jnp.float32),
                   jax.ShapeDtypeStruct((1, 1), jnp.float32)],
        scratch_shapes=[pltpu.SMEM((2,), jnp.float32)],
        compiler_params=pltpu.CompilerParams(
            dimension_semantics=("arbitrary",)),
    )(g, ef, we, bias, a2w, b2f)


def _weights_body(score_ref, c_ref, w_ref):
    p = jnp.exp(score_ref[0] - c_ref[0, 0])
    p8 = jnp.broadcast_to(p, (8, EBLK))
    pt = jnp.transpose(p8, (1, 0))
    w_ref[...] = jnp.concatenate([pt, pt], axis=1)


def _weights(score3, c):
    return pl.pallas_call(
        _weights_body,
        grid=(NEBLK,),
        in_specs=[
            pl.BlockSpec((1, 1, EBLK), lambda i: (i, 0, 0)),
            pl.BlockSpec(memory_space=pltpu.SMEM, block_shape=(1, 1),
                         index_map=lambda i: (0, 0)),
        ],
        out_specs=pl.BlockSpec((EBLK, 16), lambda i: (i, 0)),
        out_shape=jax.ShapeDtypeStruct((E, 16), jnp.float32),
    )(score3, c)


def _scatter(src, wrep, v):
    k = pl.kernel(
        _scatter_body,
        mesh=_sc_mesh(),
        out_type=jax.ShapeDtypeStruct((NC, NPAD, D), jnp.float32),
        scratch_types=[
            pltpu.VMEM((DCH,), jnp.int32),
            pltpu.VMEM((DCH,), jnp.int32),
            pltpu.VMEM((DCH, 16), jnp.float32),
            pltpu.VMEM((DCH, 16), jnp.float32),
            pltpu.VMEM((DCH, D), jnp.float32),
            pltpu.VMEM((DCH, D), jnp.float32),
            pltpu.VMEM((ZROWS, D), jnp.float32),
            pltpu.VMEM_SHARED((NPAD, D), jnp.float32),
            pltpu.SemaphoreType.DMA,
            pltpu.SemaphoreType.DMA,
            pltpu.SemaphoreType.DMA,
            pltpu.SemaphoreType.DMA,
        ],
    )
    return k(src, wrep, v)


def _scatter_body(src_hbm, wrep_hbm, v_hbm, out_hbm,
                  ib0, ib1, wb0, wb1, rb0, rb1, zbuf, acc,
                  is0, is1, g0, g1):
    cid = lax.axis_index("c")
    sid = lax.axis_index("s")
    wid = sid * NC + cid
    base0 = wid * EPW
    ibuf = (ib0, ib1)
    wbuf = (wb0, wb1)
    rbuf = (rb0, rb1)
    isem = (is0, is1)
    gsem = (g0, g1)

    def zrow(r, carry):
        for g in range(D // 16):
            zbuf[r, pl.ds(g * 16, 16)] = jnp.zeros((16,), jnp.float32)
        return carry

    lax.fori_loop(0, ZROWS, zrow, 0)
    for j in range(NPT // ZROWS):
        pltpu.sync_copy(zbuf, acc.at[pl.ds(sid * NPT + j * ZROWS, ZROWS)])
    plsc.subcore_barrier()

    def issue_is(ci, b):
        sl = pl.ds(base0 + ci * DCH, DCH)
        pltpu.async_copy(src_hbm.at[sl], ibuf[b], isem[b])
        pltpu.async_copy(wrep_hbm.at[sl], wbuf[b], isem[b])

    def wait_is(b):
        pltpu.make_async_copy(src_hbm.at[pl.ds(0, DCH)], ibuf[b],
                              isem[b]).wait()
        pltpu.make_async_copy(wrep_hbm.at[pl.ds(0, DCH)], wbuf[b],
                              isem[b]).wait()

    def issue_g(b):
        pltpu.async_copy(v_hbm.at[ibuf[b]], rbuf[b], gsem[b])

    def wait_g(b):
        pltpu.make_async_copy(v_hbm.at[ibuf[b]], rbuf[b], gsem[b]).wait()

    def work(ci, b):
        def scale_e(e, c2):
            wj = wbuf[b][e, pl.ds(0, 16)]
            for r in range(D // 16):
                sl = pl.ds(r * 16, 16)
                rbuf[b][e, sl] = rbuf[b][e, sl] * wj
            return c2

        lax.fori_loop(0, DCH, scale_e, 0)
        pltpu.sync_copy(rbuf[b], acc.at[ibuf[b]], add=True)

        @pl.when(ci + 2 < NDCH)
        def _():
            issue_is(ci + 2, b)

    issue_is(0, 0)
    wait_is(0)
    issue_g(0)
    issue_is(1, 1)

    def step(i, carry):
        for b in range(2):
            ci = 2 * i + b
            wait_g(b)
            wait_is((b + 1) % 2)
            issue_g((b + 1) % 2)
            work(ci, b)
        return carry

    lax.fori_loop(0, (NDCH - 1) // 2, step, 0)
    wait_g((NDCH - 1) % 2)
    work(NDCH - 1, (NDCH - 1) % 2)

    plsc.subcore_barrier()
    pltpu.sync_copy(acc.at[pl.ds(sid * NPT, NPT)],
                    out_hbm.at[cid, pl.ds(sid * NPT, NPT)])


def _final_body(x_ref, part_ref, wn_ref, bn_ref, gam_ref, bet_ref, out_ref):
    x = _dot(x_ref[...], wn_ref[...], ((1,), (0,))) + bn_ref[...]
    x = x + part_ref[0] + part_ref[1]
    mu = jnp.mean(x, axis=1, keepdims=True)
    xc = x - mu
    var = jnp.mean(xc * xc, axis=1, keepdims=True)
    xn = xc * lax.rsqrt(var + 1e-5)
    out_ref[...] = xn * gam_ref[...] + bet_ref[...]


def _final(x, part, wn, bn, gam, bet):
    blk = lambda i: (i, 0)
    full = lambda i: (0, 0)
    return pl.pallas_call(
        _final_body,
        grid=(NRBLK,),
        in_specs=[
            pl.BlockSpec((RBLK, D), blk),
            pl.BlockSpec((NC, RBLK, D), lambda i: (0, i, 0)),
            pl.BlockSpec((D, D), full),
            pl.BlockSpec((1, D), full),
            pl.BlockSpec((1, D), full),
            pl.BlockSpec((1, D), full),
        ],
        out_specs=pl.BlockSpec((RBLK, D), blk),
        out_shape=jax.ShapeDtypeStruct((N, D), jnp.float32),
    )(x, part, wn, bn, gam, bet)


def kernel(node_features, edge_index, edge_features, W_node_w, W_node_b,
           W_query_w, W_query_b, W_key_w, W_key_b, W_value_w, W_value_b,
           W_edge_w, W_edge_b, attn1_w, attn1_b, attn2_w, attn2_b,
           ln_gamma, ln_beta):
    src = edge_index[0]
    tgt = edge_index[1]

    a1t = attn1_w.T
    aq, ak, ae = a1t[:D], a1t[D:2 * D], a1t[2 * D:]
    wq = W_query_w.T @ aq
    wk = W_key_w.T @ ak
    we = W_edge_w.T @ ae
    bias_tot = (attn1_b + W_query_b @ aq + W_key_b @ ak + W_edge_b @ ae)
    b2f = jnp.broadcast_to(attn2_b[:, None], (H, D))

    xs, xt, v = _node_tables(node_features, wq, wk, W_value_w.T,
                             W_value_b.reshape(1, D))
    g = _edge_gather(src, tgt, xs, xt)
    score3, c = _scores(g, edge_features, we, bias_tot.reshape(1, D),
                        attn2_w, b2f)
    wrep = _weights(score3, c)
    part = _scatter(src, wrep, v)
    return _final(node_features, part, W_node_w.T, W_node_b.reshape(1, D),
                  ln_gamma.reshape(1, D), ln_beta.reshape(1, D))


# speedup vs baseline: 5.5278x; 1.7663x over previous
import functools

import jax
import jax.numpy as jnp
from jax import lax
from jax.experimental import pallas as pl
from jax.experimental.pallas import tpu as pltpu
from jax.experimental.pallas import tpu_sc as plsc

N, E, D, DE, H = 10000, 320000, 128, 16, 8
NC, NS = 2, 16
NW = NC * NS
EPW = E // NW
BCH = 200
NBCH = EPW // BCH
DCH = 40
NDCH = EPW // DCH
EBLK = 12800
NEBLK = E // EBLK
RBLK = 400
NRBLK = N // RBLK
NPAD = 10240
NPT = NPAD // NS
ZROWS = 16

_HI = jax.lax.Precision.HIGHEST


def _dot(a, b, dims):
    return lax.dot_general(a, b, (dims, ((), ())), precision=_HI,
                           preferred_element_type=jnp.float32)


def _lrelu(x):
    return jnp.maximum(x, 0.2 * x)


def _node_tables_body(x_ref, wq_ref, wk_ref, wv_ref, bv_ref,
                      xs_ref, xt_ref, v_ref):
    x = x_ref[...]
    xs_ref[...] = _dot(x, wq_ref[...], ((1,), (0,)))
    xt_ref[...] = _dot(x, wk_ref[...], ((1,), (0,)))
    v_ref[...] = _dot(x, wv_ref[...], ((1,), (0,))) + bv_ref[...]


def _node_tables(x, wq, wk, wv, bv):
    blk = lambda i: (i, 0)
    full = lambda i: (0, 0)
    return pl.pallas_call(
        _node_tables_body,
        grid=(NRBLK,),
        in_specs=[
            pl.BlockSpec((RBLK, D), blk),
            pl.BlockSpec((D, D), full),
            pl.BlockSpec((D, D), full),
            pl.BlockSpec((D, D), full),
            pl.BlockSpec((1, D), full),
        ],
        out_specs=[
            pl.BlockSpec((RBLK, D), blk),
            pl.BlockSpec((RBLK, D), blk),
            pl.BlockSpec((RBLK, D), blk),
        ],
        out_shape=[jax.ShapeDtypeStruct((N, D), jnp.float32)] * 3,
    )(x, wq, wk, wv, bv)


@functools.cache
def _sc_mesh():
    return plsc.VectorSubcoreMesh(core_axis_name="c", subcore_axis_name="s",
                                  num_cores=NC, num_subcores=NS)


def _edge_gather(src, tgt, xs, xt):
    k = pl.kernel(
        _edge_gather_body,
        mesh=_sc_mesh(),
        out_type=jax.ShapeDtypeStruct((E, D), jnp.float32),
        scratch_types=[
            pltpu.VMEM((EPW,), jnp.int32),
            pltpu.VMEM((EPW,), jnp.int32),
            pltpu.VMEM((BCH, D), jnp.float32),
            pltpu.VMEM((BCH, D), jnp.float32),
            pltpu.VMEM((BCH, D), jnp.float32),
            pltpu.VMEM((BCH, D), jnp.float32),
            pltpu.SemaphoreType.DMA,
            pltpu.SemaphoreType.DMA,
            pltpu.SemaphoreType.DMA,
            pltpu.SemaphoreType.DMA,
        ],
    )
    return k(src, tgt, xs, xt)


def _edge_gather_body(src_hbm, tgt_hbm, xs_hbm, xt_hbm, g_hbm,
                      idx_s, idx_t, ra0, rb0, ra1, rb1,
                      s0, s1, s2, s3):
    wid = lax.axis_index("s") * NC + lax.axis_index("c")
    base0 = wid * EPW
    abuf = (ra0, ra1)
    bbuf = (rb0, rb1)
    asem = (s0, s1)
    bsem = (s2, s3)

    pltpu.sync_copy(src_hbm.at[pl.ds(base0, EPW)], idx_s)
    pltpu.sync_copy(tgt_hbm.at[pl.ds(base0, EPW)], idx_t)

    def issue(ci, b):
        sl = pl.ds(ci * BCH, BCH)
        pltpu.async_copy(xs_hbm.at[idx_s.at[sl]], abuf[b], asem[b])
        pltpu.async_copy(xt_hbm.at[idx_t.at[sl]], bbuf[b], bsem[b])

    def wait(b):
        pltpu.make_async_copy(xs_hbm.at[idx_s.at[pl.ds(0, BCH)]],
                              abuf[b], asem[b]).wait()
        pltpu.make_async_copy(xt_hbm.at[idx_t.at[pl.ds(0, BCH)]],
                              bbuf[b], bsem[b]).wait()

    issue(0, 0)

    def step(i, carry):
        for b in range(2):
            ci = 2 * i + b

            @pl.when(ci + 1 < NBCH)
            def _():
                issue(ci + 1, (b + 1) % 2)

            wait(b)

            def add_row(e, c2):
                for g in range(D // 16):
                    sl = pl.ds(g * 16, 16)
                    abuf[b][e, sl] = abuf[b][e, sl] + bbuf[b][e, sl]
                return c2

            lax.fori_loop(0, BCH, add_row, 0)
            pltpu.sync_copy(abuf[b], g_hbm.at[pl.ds(base0 + ci * BCH, BCH)])
        return carry

    lax.fori_loop(0, NBCH // 2, step, 0)


def _sdot(a, b, dims):
    return lax.dot_general(a, b, (dims, ((), ())),
                           preferred_element_type=jnp.float32)


def _scores_body(g_ref, ef_ref, we_ref, bias_ref, a2w_ref, b2_ref,
                 score_ref, c_ref, mz_ref):
    i = pl.program_id(0)
    t = g_ref[...] + _sdot(ef_ref[...], we_ref[...], ((1,), (0,))) + bias_ref[...]
    a = _lrelu(t)
    st = _sdot(a2w_ref[...], a, ((1,), (1,)))
    st = _lrelu(st + jnp.broadcast_to(b2_ref[:, 0:1], (H, EBLK)))
    srow = jnp.mean(st, axis=0, keepdims=True)
    score_ref[...] = srow[None]

    m_old = jnp.where(i == 0, -1e30, mz_ref[0])
    z_old = jnp.where(i == 0, 0.0, mz_ref[1])
    m_blk = jnp.max(srow)
    m_new = jnp.maximum(m_old, m_blk)
    z_new = z_old * jnp.exp(m_old - m_new) + jnp.sum(jnp.exp(srow - m_new))
    mz_ref[0] = m_new
    mz_ref[1] = z_new

    @pl.when(i == NEBLK - 1)
    def _():
        c_ref[0, 0] = m_new + jnp.log(z_new)


def _scores(g, ef, we, bias, a2w, b2f):
    blk = lambda i: (i, 0)
    full = lambda i: (0, 0)
    return pl.pallas_call(
        _scores_body,
        grid=(NEBLK,),
        in_specs=[
            pl.BlockSpec((EBLK, D), blk),
            pl.BlockSpec((EBLK, DE), blk),
            pl.BlockSpec((DE, D), full),
            pl.BlockSpec((1, D), full),
            pl.BlockSpec((H, D), full),
            pl.BlockSpec((H, D), full),
        ],
        out_specs=[
            pl.BlockSpec((1, 1, EBLK), lambda i: (i, 0, 0)),
            pl.BlockSpec(memory_space=pltpu.SMEM, block_shape=(1, 1),
                         index_map=full),
        ],
        out_shape=[jax.ShapeDtypeStruct((NEBLK, 1, EBLK), jnp.float32),
                   jax.ShapeDtypeStruct((1, 1), jnp.float32)],
        scratch_shapes=[pltpu.SMEM((2,), jnp.float32)],
        compiler_params=pltpu.CompilerParams(
            dimension_semantics=("arbitrary",)),
    )(g, ef, we, bias, a2w, b2f)


def _weights_body(score_ref, c_ref, w_ref):
    p = jnp.exp(score_ref[0] - c_ref[0, 0])
    p8 = jnp.broadcast_to(p, (8, EBLK))
    pt = jnp.transpose(p8, (1, 0))
    w_ref[...] = jnp.concatenate([pt, pt], axis=1)


def _weights(score3, c):
    return pl.pallas_call(
        _weights_body,
        grid=(NEBLK,),
        in_specs=[
            pl.BlockSpec((1, 1, EBLK), lambda i: (i, 0, 0)),
            pl.BlockSpec(memory_space=pltpu.SMEM, block_shape=(1, 1),
                         index_map=lambda i: (0, 0)),
        ],
        out_specs=pl.BlockSpec((EBLK, 16), lambda i: (i, 0)),
        out_shape=jax.ShapeDtypeStruct((E, 16), jnp.float32),
    )(score3, c)


def _scatter(src, wrep, v):
    k = pl.kernel(
        _scatter_body,
        mesh=_sc_mesh(),
        out_type=jax.ShapeDtypeStruct((NC, NPAD, D), jnp.float32),
        scratch_types=(
            [pltpu.VMEM((DCH,), jnp.int32)] * 4
            + [pltpu.VMEM((DCH, 16), jnp.float32)] * 4
            + [pltpu.VMEM((DCH, D), jnp.float32)] * 4
            + [pltpu.VMEM((ZROWS, D), jnp.float32),
               pltpu.VMEM_SHARED((NPAD, D), jnp.float32)]
            + [pltpu.SemaphoreType.DMA] * 8
        ),
    )
    return k(src, wrep, v)


def _scatter_body(src_hbm, wrep_hbm, v_hbm, out_hbm,
                  ib0, ib1, ib2, ib3, wb0, wb1, wb2, wb3,
                  rb0, rb1, rb2, rb3, zbuf, acc,
                  is0, is1, is2, is3, g0, g1, g2, g3):
    cid = lax.axis_index("c")
    sid = lax.axis_index("s")
    wid = sid * NC + cid
    base0 = wid * EPW
    ibuf = (ib0, ib1, ib2, ib3)
    wbuf = (wb0, wb1, wb2, wb3)
    rbuf = (rb0, rb1, rb2, rb3)
    isem = (is0, is1, is2, is3)
    gsem = (g0, g1, g2, g3)

    def zrow(r, carry):
        for g in range(D // 16):
            zbuf[r, pl.ds(g * 16, 16)] = jnp.zeros((16,), jnp.float32)
        return carry

    lax.fori_loop(0, ZROWS, zrow, 0)
    for j in range(NPT // ZROWS):
        pltpu.sync_copy(zbuf, acc.at[pl.ds(sid * NPT + j * ZROWS, ZROWS)])
    plsc.subcore_barrier()

    def issue_is(ci, b):
        sl = pl.ds(base0 + ci * DCH, DCH)
        pltpu.async_copy(src_hbm.at[sl], ibuf[b], isem[b])
        pltpu.async_copy(wrep_hbm.at[sl], wbuf[b], isem[b])

    def wait_is(b):
        pltpu.make_async_copy(src_hbm.at[pl.ds(0, DCH)], ibuf[b],
                              isem[b]).wait()
        pltpu.make_async_copy(wrep_hbm.at[pl.ds(0, DCH)], wbuf[b],
                              isem[b]).wait()

    def issue_g(b):
        pltpu.async_copy(v_hbm.at[ibuf[b]], rbuf[b], gsem[b])

    def wait_g(b):
        pltpu.make_async_copy(v_hbm.at[ibuf[b]], rbuf[b], gsem[b]).wait()

    def work(ci, b):
        @pl.when(ci + 2 < NDCH)
        def _():
            wait_is((b + 2) % 4)
            issue_g((b + 2) % 4)

        def scale_e(e, c2):
            wj = wbuf[b][e, pl.ds(0, 16)]
            for r in range(D // 16):
                sl = pl.ds(r * 16, 16)
                rbuf[b][e, sl] = rbuf[b][e, sl] * wj
            return c2

        lax.fori_loop(0, DCH, scale_e, 0)
        pltpu.sync_copy(rbuf[b], acc.at[ibuf[b]], add=True)

        @pl.when(ci + 4 < NDCH)
        def _():
            issue_is(ci + 4, b)

    issue_is(0, 0)
    issue_is(1, 1)
    wait_is(0)
    issue_g(0)
    issue_is(2, 2)
    wait_is(1)
    issue_g(1)
    issue_is(3, 3)

    def step(i, carry):
        for b in range(4):
            ci = 4 * i + b
            wait_g(b)
            work(ci, b)
        return carry

    main = (NDCH // 4) * 4
    lax.fori_loop(0, NDCH // 4, step, 0)
    for ci in range(main, NDCH):
        wait_g(ci % 4)
        work(ci, ci % 4)

    plsc.subcore_barrier()
    pltpu.sync_copy(acc.at[pl.ds(sid * NPT, NPT)],
                    out_hbm.at[cid, pl.ds(sid * NPT, NPT)])


def _final_body(x_ref, part_ref, wn_ref, bn_ref, gam_ref, bet_ref, out_ref):
    x = _dot(x_ref[...], wn_ref[...], ((1,), (0,))) + bn_ref[...]
    x = x + part_ref[0] + part_ref[1]
    mu = jnp.mean(x, axis=1, keepdims=True)
    xc = x - mu
    var = jnp.mean(xc * xc, axis=1, keepdims=True)
    xn = xc * lax.rsqrt(var + 1e-5)
    out_ref[...] = xn * gam_ref[...] + bet_ref[...]


def _final(x, part, wn, bn, gam, bet):
    blk = lambda i: (i, 0)
    full = lambda i: (0, 0)
    return pl.pallas_call(
        _final_body,
        grid=(NRBLK,),
        in_specs=[
            pl.BlockSpec((RBLK, D), blk),
            pl.BlockSpec((NC, RBLK, D), lambda i: (0, i, 0)),
            pl.BlockSpec((D, D), full),
            pl.BlockSpec((1, D), full),
            pl.BlockSpec((1, D), full),
            pl.BlockSpec((1, D), full),
        ],
        out_specs=pl.BlockSpec((RBLK, D), blk),
        out_shape=jax.ShapeDtypeStruct((N, D), jnp.float32),
    )(x, part, wn, bn, gam, bet)


def kernel(node_features, edge_index, edge_features, W_node_w, W_node_b,
           W_query_w, W_query_b, W_key_w, W_key_b, W_value_w, W_value_b,
           W_edge_w, W_edge_b, attn1_w, attn1_b, attn2_w, attn2_b,
           ln_gamma, ln_beta):
    src = edge_index[0]
    tgt = edge_index[1]

    a1t = attn1_w.T
    aq, ak, ae = a1t[:D], a1t[D:2 * D], a1t[2 * D:]
    wq = W_query_w.T @ aq
    wk = W_key_w.T @ ak
    we = W_edge_w.T @ ae
    bias_tot = (attn1_b + W_query_b @ aq + W_key_b @ ak + W_edge_b @ ae)
    b2f = jnp.broadcast_to(attn2_b[:, None], (H, D))

    xs, xt, v = _node_tables(node_features, wq, wk, W_value_w.T,
                             W_value_b.reshape(1, D))
    g = _edge_gather(src, tgt, xs, xt)
    score3, c = _scores(g, edge_features, we, bias_tot.reshape(1, D),
                        attn2_w, b2f)
    wrep = _weights(score3, c)
    part = _scatter(src, wrep, v)
    return _final(node_features, part, W_node_w.T, W_node_b.reshape(1, D),
                  ln_gamma.reshape(1, D), ln_beta.reshape(1, D))
